# baseline (speedup 1.0000x reference)
import functools

import jax
import jax.numpy as jnp
from jax import lax
from jax.experimental import pallas as pl
from jax.experimental.pallas import tpu as pltpu
from jax.experimental.pallas import tpu_sc as plsc

N = 10000
NP = 10240
D = 128
E = 320000

NSUB = 16
EPS = E // NSUB
CH = 80
NCHUNK = EPS // CH
RPS = NP // NSUB
NSEG = 10
SEGCH = NCHUNK // NSEG
SEGE = SEGCH * CH
NBUF = 4

_f32 = jnp.float32
_i32 = jnp.int32

_mesh = plsc.VectorSubcoreMesh(core_axis_name="c", subcore_axis_name="s")


def _stage_idx(all_ref, base, buf, n):
    for k in range(n // 16):
        buf[pl.ds(16 * k, 16)] = all_ref[pl.ds(base + 16 * k, 16)]


def _agg_body(x_hbm, src0, dst0, src1, dst1, z_hbm, out0, out1,
              sidx_all, didx_all,
              si0, si1, si2, si3, di0, di1, di2, di3,
              rows0, rows1, rows2, rows3,
              accum, sem0, sem1, sem2, sem3):
    c = lax.axis_index("c")
    s = lax.axis_index("s")

    pltpu.sync_copy(z_hbm, accum.at[pl.ds(s * RPS, RPS)])

    bufs = ((si0, di0, rows0, sem0),
            (si1, di1, rows1, sem1),
            (si2, di2, rows2, sem2),
            (si3, di3, rows3, sem3))

    def do_rel(src, dst):
        plsc.subcore_barrier()

        def fire(g, b):
            si, di, rows, sem = bufs[b]
            _stage_idx(sidx_all, g * CH, si, CH)
            _stage_idx(didx_all, g * CH, di, CH)
            pltpu.async_copy(x_hbm.at[si], rows, sem)

        def drain(b):
            si, di, rows, sem = bufs[b]
            pltpu.make_async_copy(x_hbm.at[si], rows, sem).wait()
            pltpu.sync_copy(rows, accum.at[di], add=True)

        def seg_body(q, carry):
            base = s * EPS + q * SEGE
            pltpu.sync_copy(src.at[pl.ds(base, SEGE)], sidx_all)
            pltpu.sync_copy(dst.at[pl.ds(base, SEGE)], didx_all)
            fire(0, 0)
            fire(1, 1)
            fire(2, 2)

            def triple(u, carry2):
                for j in range(NBUF):
                    g = NBUF * u + j

                    @pl.when(g + 3 < SEGCH)
                    def _(g=g, j=j):
                        fire(g + 3, (j + 3) % NBUF)

                    @pl.when(g < SEGCH)
                    def _(g=g, j=j):
                        drain(j)
                return carry2

            lax.fori_loop(0, (SEGCH + NBUF - 1) // NBUF, triple, 0)
            return carry

        lax.fori_loop(0, NSEG, seg_body, 0)

    def rel0():
        do_rel(src0, dst0)

    def rel1():
        do_rel(src1, dst1)

    pl.when(c == 0)(rel0)
    pl.when(c == 1)(rel1)
    plsc.subcore_barrier()

    sl = pl.ds(s * RPS, RPS)

    @pl.when(c == 0)
    def _():
        pltpu.sync_copy(accum.at[sl], out0.at[sl])

    @pl.when(c == 1)
    def _():
        pltpu.sync_copy(accum.at[sl], out1.at[sl])


_agg_call = pl.kernel(
    _agg_body,
    out_type=(
        jax.ShapeDtypeStruct((NP, D), _f32),
        jax.ShapeDtypeStruct((NP, D), _f32),
    ),
    mesh=_mesh,
    scratch_types=(
        [pltpu.VMEM((SEGE,), _i32)] * 2
        + [pltpu.VMEM((CH,), _i32)] * 8
        + [pltpu.VMEM((CH, D), _f32)] * 4
        + [pltpu.VMEM_SHARED((NP, D), _f32)]
        + [pltpu.SemaphoreType.DMA] * 4
    ),
)


def _cnt_body(dst0, dst1, z_hbm, ones_hbm, cnt0, cnt1,
              didx_all, didx, ones_v, cntacc):
    c = lax.axis_index("c")
    s = lax.axis_index("s")

    pltpu.sync_copy(z_hbm, cntacc.at[pl.ds(s * RPS, RPS)])
    pltpu.sync_copy(ones_hbm, ones_v)

    def do_rel(dst):
        pltpu.sync_copy(dst.at[pl.ds(s * EPS, EPS)], didx_all)
        plsc.subcore_barrier()

        def chunk(g, carry):
            _stage_idx(didx_all, g * CH, didx, CH)
            pltpu.sync_copy(ones_v, cntacc.at[didx], add=True)
            return carry
        lax.fori_loop(0, NCHUNK, chunk, 0)

    def rel0():
        do_rel(dst0)

    def rel1():
        do_rel(dst1)

    pl.when(c == 0)(rel0)
    pl.when(c == 1)(rel1)
    plsc.subcore_barrier()

    sl = pl.ds(s * RPS, RPS)

    @pl.when(c == 0)
    def _():
        pltpu.sync_copy(cntacc.at[sl], cnt0.at[sl])

    @pl.when(c == 1)
    def _():
        pltpu.sync_copy(cntacc.at[sl], cnt1.at[sl])


_cnt_call = pl.kernel(
    _cnt_body,
    out_type=(
        jax.ShapeDtypeStruct((NP, D), _f32),
        jax.ShapeDtypeStruct((NP, D), _f32),
    ),
    mesh=_mesh,
    scratch_types=[
        pltpu.VMEM((EPS,), _i32),
        pltpu.VMEM((CH,), _i32),
        pltpu.VMEM((CH, D), _f32),
        pltpu.VMEM_SHARED((NP, D), _f32),
    ],
)

R = 1000


def _mm_body(relu, h_ref, a0_ref, a1_ref, c0_ref, c1_ref,
             wl0_ref, wl1_ref, wr_ref, b_ref, o_ref):
    c0 = jnp.maximum(c0_ref[...], 1.0)
    c1 = jnp.maximum(c1_ref[...], 1.0)
    m0 = a0_ref[...] / c0
    m1 = a1_ref[...] / c1
    acc = jnp.dot(m0, wl0_ref[...], preferred_element_type=_f32)
    acc = acc + jnp.dot(m1, wl1_ref[...], preferred_element_type=_f32)
    acc = acc + jnp.dot(h_ref[...], wr_ref[...], preferred_element_type=_f32)
    acc = acc + b_ref[...]
    if relu:
        acc = jnp.maximum(acc, 0.0)
    o_ref[...] = acc


def _make_mm(relu):
    row_spec = pl.BlockSpec((R, D), lambda i: (i, 0))
    cnt_spec = pl.BlockSpec((R, 1), lambda i: (i, 0))
    w_spec = pl.BlockSpec((D, D), lambda i: (0, 0))
    b_spec = pl.BlockSpec((1, D), lambda i: (0, 0))
    return pl.pallas_call(
        functools.partial(_mm_body, relu),
        grid=(N // R,),
        in_specs=[row_spec, row_spec, row_spec, cnt_spec, cnt_spec,
                  w_spec, w_spec, w_spec, b_spec],
        out_specs=row_spec,
        out_shape=jax.ShapeDtypeStruct((N, D), _f32),
    )


_mm_relu = _make_mm(True)
_mm_plain = _make_mm(False)


def kernel(x, edge_index_rel0, edge_index_rel1,
           W_l_0_0, b_l_0_0, W_r_0_0, W_l_0_1, b_l_0_1, W_r_0_1,
           W_l_1_0, b_l_1_0, W_r_1_0, W_l_1_1, b_l_1_1, W_r_1_1):
    zeros = jnp.zeros((RPS, D), _f32)
    ones = jnp.ones((CH, D), _f32)

    s0, d0 = edge_index_rel0[0], edge_index_rel0[1]
    s1, d1 = edge_index_rel1[0], edge_index_rel1[1]

    cnt0, cnt1 = _cnt_call(d0, d1, zeros, ones)
    c0 = cnt0[:N, :1]
    c1 = cnt1[:N, :1]

    a0, a1 = _agg_call(x, s0, d0, s1, d1, zeros)
    h1 = _mm_relu(x, a0[:N], a1[:N], c0, c1,
                  W_l_0_0, W_l_0_1, W_r_0_0 + W_r_0_1,
                  (b_l_0_0 + b_l_0_1)[None, :])

    a0, a1 = _agg_call(h1, s0, d0, s1, d1, zeros)
    out = _mm_plain(h1, a0[:N], a1[:N], c0, c1,
                    W_l_1_0, W_l_1_1, W_r_1_0 + W_r_1_1,
                    (b_l_1_0 + b_l_1_1)[None, :])
    return out


# speedup vs baseline: 1.0537x; 1.0537x over previous
import functools

import jax
import jax.numpy as jnp
from jax import lax
from jax.experimental import pallas as pl
from jax.experimental.pallas import tpu as pltpu
from jax.experimental.pallas import tpu_sc as plsc

N = 10000
NP = 10240
D = 128
E = 320000

NSUB = 16
EPS = E // NSUB
CH = 80
NCHUNK = EPS // CH
RPS = NP // NSUB
NSEG = 5
SEGCH = NCHUNK // NSEG
SEGE = SEGCH * CH
NBUF = 3

_f32 = jnp.float32
_i32 = jnp.int32

_mesh = plsc.VectorSubcoreMesh(core_axis_name="c", subcore_axis_name="s")


def _stage_idx(all_ref, base, buf, n):
    for k in range(n // 16):
        buf[pl.ds(16 * k, 16)] = all_ref[pl.ds(base + 16 * k, 16)]


def _agg_body(x_hbm, src0, dst0, src1, dst1, z_hbm, out0, out1,
              sidx_a, didx_a, sidx_b, didx_b,
              si0, si1, si2, di0, di1, di2,
              rows0, rows1, rows2,
              accum, sem0, sem1, sem2, semi):
    c = lax.axis_index("c")
    s = lax.axis_index("s")

    pltpu.sync_copy(z_hbm, accum.at[pl.ds(s * RPS, RPS)])

    bufs = ((si0, di0, rows0, sem0),
            (si1, di1, rows1, sem1),
            (si2, di2, rows2, sem2))

    seg_bufs = ((sidx_a, didx_a), (sidx_b, didx_b))

    def do_rel(src, dst):
        plsc.subcore_barrier()

        def seg_fetch(q, sb):
            sall, dall = sb
            base = s * EPS + q * SEGE
            pltpu.async_copy(src.at[pl.ds(base, SEGE)], sall, semi)
            pltpu.async_copy(dst.at[pl.ds(base, SEGE)], dall, semi)

        def seg_wait(q, sb):
            sall, dall = sb
            base = s * EPS + q * SEGE
            pltpu.make_async_copy(src.at[pl.ds(base, SEGE)], sall, semi).wait()
            pltpu.make_async_copy(dst.at[pl.ds(base, SEGE)], dall, semi).wait()

        seg_fetch(0, seg_bufs[0])
        for q in range(NSEG):
            sall, dall = seg_bufs[q % 2]
            seg_wait(q, seg_bufs[q % 2])
            if q + 1 < NSEG:
                seg_fetch(q + 1, seg_bufs[(q + 1) % 2])

            def fire(g, b):
                si, di, rows, sem = bufs[b]
                _stage_idx(sall, g * CH, si, CH)
                _stage_idx(dall, g * CH, di, CH)
                pltpu.async_copy(x_hbm.at[si], rows, sem)

            def drain(b):
                si, di, rows, sem = bufs[b]
                pltpu.make_async_copy(x_hbm.at[si], rows, sem).wait()
                pltpu.sync_copy(rows, accum.at[di], add=True)

            fire(0, 0)
            fire(1, 1)

            def triple(u, carry2, fire=fire, drain=drain):
                for j in range(NBUF):
                    g = NBUF * u + j

                    @pl.when(g + 2 < SEGCH)
                    def _(g=g, j=j):
                        fire(g + 2, (j + 2) % NBUF)

                    @pl.when(g < SEGCH)
                    def _(g=g, j=j):
                        drain(j)
                return carry2

            lax.fori_loop(0, (SEGCH + NBUF - 1) // NBUF, triple, 0)

    def rel0():
        do_rel(src0, dst0)

    def rel1():
        do_rel(src1, dst1)

    pl.when(c == 0)(rel0)
    pl.when(c == 1)(rel1)
    plsc.subcore_barrier()

    sl = pl.ds(s * RPS, RPS)

    @pl.when(c == 0)
    def _():
        pltpu.sync_copy(accum.at[sl], out0.at[sl])

    @pl.when(c == 1)
    def _():
        pltpu.sync_copy(accum.at[sl], out1.at[sl])


_agg_call = pl.kernel(
    _agg_body,
    out_type=(
        jax.ShapeDtypeStruct((NP, D), _f32),
        jax.ShapeDtypeStruct((NP, D), _f32),
    ),
    mesh=_mesh,
    scratch_types=(
        [pltpu.VMEM((SEGE,), _i32)] * 4
        + [pltpu.VMEM((CH,), _i32)] * 6
        + [pltpu.VMEM((CH, D), _f32)] * 3
        + [pltpu.VMEM_SHARED((NP, D), _f32)]
        + [pltpu.SemaphoreType.DMA] * 4
    ),
)


def _cnt_body(dst0, dst1, z_hbm, ones_hbm, cnt0, cnt1,
              didx_all, didx, ones_v, cntacc):
    c = lax.axis_index("c")
    s = lax.axis_index("s")

    pltpu.sync_copy(z_hbm, cntacc.at[pl.ds(s * RPS, RPS)])
    pltpu.sync_copy(ones_hbm, ones_v)

    def do_rel(dst):
        pltpu.sync_copy(dst.at[pl.ds(s * EPS, EPS)], didx_all)
        plsc.subcore_barrier()

        def chunk(g, carry):
            _stage_idx(didx_all, g * CH, didx, CH)
            pltpu.sync_copy(ones_v, cntacc.at[didx], add=True)
            return carry
        lax.fori_loop(0, NCHUNK, chunk, 0)

    def rel0():
        do_rel(dst0)

    def rel1():
        do_rel(dst1)

    pl.when(c == 0)(rel0)
    pl.when(c == 1)(rel1)
    plsc.subcore_barrier()

    sl = pl.ds(s * RPS, RPS)

    @pl.when(c == 0)
    def _():
        pltpu.sync_copy(cntacc.at[sl], cnt0.at[sl])

    @pl.when(c == 1)
    def _():
        pltpu.sync_copy(cntacc.at[sl], cnt1.at[sl])


_cnt_call = pl.kernel(
    _cnt_body,
    out_type=(
        jax.ShapeDtypeStruct((NP, D), _f32),
        jax.ShapeDtypeStruct((NP, D), _f32),
    ),
    mesh=_mesh,
    scratch_types=[
        pltpu.VMEM((EPS,), _i32),
        pltpu.VMEM((CH,), _i32),
        pltpu.VMEM((CH, D), _f32),
        pltpu.VMEM_SHARED((NP, D), _f32),
    ],
)

R = 1000


def _mm_body(relu, h_ref, a0_ref, a1_ref, c0_ref, c1_ref,
             wl0_ref, wl1_ref, wr_ref, b_ref, o_ref):
    c0 = jnp.maximum(c0_ref[...], 1.0)
    c1 = jnp.maximum(c1_ref[...], 1.0)
    m0 = a0_ref[...] / c0
    m1 = a1_ref[...] / c1
    acc = jnp.dot(m0, wl0_ref[...], preferred_element_type=_f32)
    acc = acc + jnp.dot(m1, wl1_ref[...], preferred_element_type=_f32)
    acc = acc + jnp.dot(h_ref[...], wr_ref[...], preferred_element_type=_f32)
    acc = acc + b_ref[...]
    if relu:
        acc = jnp.maximum(acc, 0.0)
    o_ref[...] = acc


def _make_mm(relu):
    row_spec = pl.BlockSpec((R, D), lambda i: (i, 0))
    cnt_spec = pl.BlockSpec((R, 1), lambda i: (i, 0))
    w_spec = pl.BlockSpec((D, D), lambda i: (0, 0))
    b_spec = pl.BlockSpec((1, D), lambda i: (0, 0))
    return pl.pallas_call(
        functools.partial(_mm_body, relu),
        grid=(N // R,),
        in_specs=[row_spec, row_spec, row_spec, cnt_spec, cnt_spec,
                  w_spec, w_spec, w_spec, b_spec],
        out_specs=row_spec,
        out_shape=jax.ShapeDtypeStruct((N, D), _f32),
    )


_mm_relu = _make_mm(True)
_mm_plain = _make_mm(False)


def kernel(x, edge_index_rel0, edge_index_rel1,
           W_l_0_0, b_l_0_0, W_r_0_0, W_l_0_1, b_l_0_1, W_r_0_1,
           W_l_1_0, b_l_1_0, W_r_1_0, W_l_1_1, b_l_1_1, W_r_1_1):
    zeros = jnp.zeros((RPS, D), _f32)
    ones = jnp.ones((CH, D), _f32)

    s0, d0 = edge_index_rel0[0], edge_index_rel0[1]
    s1, d1 = edge_index_rel1[0], edge_index_rel1[1]

    cnt0, cnt1 = _cnt_call(d0, d1, zeros, ones)
    c0 = cnt0[:N, :1]
    c1 = cnt1[:N, :1]

    a0, a1 = _agg_call(x, s0, d0, s1, d1, zeros)
    h1 = _mm_relu(x, a0[:N], a1[:N], c0, c1,
                  W_l_0_0, W_l_0_1, W_r_0_0 + W_r_0_1,
                  (b_l_0_0 + b_l_0_1)[None, :])

    a0, a1 = _agg_call(h1, s0, d0, s1, d1, zeros)
    out = _mm_plain(h1, a0[:N], a1[:N], c0, c1,
                    W_l_1_0, W_l_1_1, W_r_1_0 + W_r_1_1,
                    (b_l_1_0 + b_l_1_1)[None, :])
    return out


# speedup vs baseline: 1.0658x; 1.0114x over previous
import functools

import jax
import jax.numpy as jnp
from jax import lax
from jax.experimental import pallas as pl
from jax.experimental.pallas import tpu as pltpu
from jax.experimental.pallas import tpu_sc as plsc

N = 10000
NP = 10240
D = 128
E = 320000

NSUB = 16
EPS = E // NSUB
CH = 80
NCHUNK = EPS // CH
RPS = NP // NSUB
NSEG = 5
SEGCH = NCHUNK // NSEG
SEGE = SEGCH * CH
NBUF = 3

_f32 = jnp.float32
_i32 = jnp.int32

_mesh = plsc.VectorSubcoreMesh(core_axis_name="c", subcore_axis_name="s")


def _stage_idx(all_ref, base, buf, n):
    for k in range(n // 16):
        buf[pl.ds(16 * k, 16)] = all_ref[pl.ds(base + 16 * k, 16)]


def _agg_body(x_hbm, src0, dst0, src1, dst1, z_hbm, out0, out1,
              sidx_a, didx_a, sidx_b, didx_b,
              si0, si1, si2, di0, di1, di2,
              rows0, rows1, rows2,
              accum, sem0, sem1, sem2, semi, ssc0, ssc1, ssc2):
    c = lax.axis_index("c")
    s = lax.axis_index("s")

    pltpu.sync_copy(z_hbm, accum.at[pl.ds(s * RPS, RPS)])

    bufs = ((si0, di0, rows0, sem0, ssc0),
            (si1, di1, rows1, sem1, ssc1),
            (si2, di2, rows2, sem2, ssc2))

    seg_bufs = ((sidx_a, didx_a), (sidx_b, didx_b))

    def do_rel(src, dst):
        plsc.subcore_barrier()

        def seg_fetch(q, sb):
            sall, dall = sb
            base = s * EPS + q * SEGE
            pltpu.async_copy(src.at[pl.ds(base, SEGE)], sall, semi)
            pltpu.async_copy(dst.at[pl.ds(base, SEGE)], dall, semi)

        def seg_wait(q, sb):
            sall, dall = sb
            base = s * EPS + q * SEGE
            pltpu.make_async_copy(src.at[pl.ds(base, SEGE)], sall, semi).wait()
            pltpu.make_async_copy(dst.at[pl.ds(base, SEGE)], dall, semi).wait()

        seg_fetch(0, seg_bufs[0])
        for q in range(NSEG):
            sall, dall = seg_bufs[q % 2]
            seg_wait(q, seg_bufs[q % 2])
            if q + 1 < NSEG:
                seg_fetch(q + 1, seg_bufs[(q + 1) % 2])

            def fire(g, b, wait_scatter):
                si, di, rows, sem, ssc = bufs[b]
                if wait_scatter:
                    pltpu.make_async_copy(rows, accum.at[di], ssc).wait()
                _stage_idx(sall, g * CH, si, CH)
                _stage_idx(dall, g * CH, di, CH)
                pltpu.async_copy(x_hbm.at[si], rows, sem)

            def drain(b):
                si, di, rows, sem, ssc = bufs[b]
                pltpu.make_async_copy(x_hbm.at[si], rows, sem).wait()
                pltpu.async_copy(rows, accum.at[di], ssc, add=True)

            for j in range(NBUF):
                fire(j, j, q > 0)

            def triple(u, carry2, fire=fire, drain=drain):
                for j in range(NBUF):
                    g = NBUF * u + j

                    @pl.when(g < SEGCH)
                    def _(g=g, j=j):
                        drain(j)

                    @pl.when(g + NBUF < SEGCH)
                    def _(g=g, j=j):
                        fire(g + NBUF, j, True)
                return carry2

            lax.fori_loop(0, (SEGCH + NBUF - 1) // NBUF, triple, 0)

        for j in range(NBUF):
            si, di, rows, sem, ssc = bufs[j]
            pltpu.make_async_copy(rows, accum.at[di], ssc).wait()

    def rel0():
        do_rel(src0, dst0)

    def rel1():
        do_rel(src1, dst1)

    pl.when(c == 0)(rel0)
    pl.when(c == 1)(rel1)
    plsc.subcore_barrier()

    sl = pl.ds(s * RPS, RPS)

    @pl.when(c == 0)
    def _():
        pltpu.sync_copy(accum.at[sl], out0.at[sl])

    @pl.when(c == 1)
    def _():
        pltpu.sync_copy(accum.at[sl], out1.at[sl])


_agg_call = pl.kernel(
    _agg_body,
    out_type=(
        jax.ShapeDtypeStruct((NP, D), _f32),
        jax.ShapeDtypeStruct((NP, D), _f32),
    ),
    mesh=_mesh,
    scratch_types=(
        [pltpu.VMEM((SEGE,), _i32)] * 4
        + [pltpu.VMEM((CH,), _i32)] * 6
        + [pltpu.VMEM((CH, D), _f32)] * 3
        + [pltpu.VMEM_SHARED((NP, D), _f32)]
        + [pltpu.SemaphoreType.DMA] * 7
    ),
)


def _cnt_body(dst0, dst1, z_hbm, ones_hbm, cnt0, cnt1,
              didx_all, didx, ones_v, cntacc):
    c = lax.axis_index("c")
    s = lax.axis_index("s")

    pltpu.sync_copy(z_hbm, cntacc.at[pl.ds(s * RPS, RPS)])
    pltpu.sync_copy(ones_hbm, ones_v)

    def do_rel(dst):
        pltpu.sync_copy(dst.at[pl.ds(s * EPS, EPS)], didx_all)
        plsc.subcore_barrier()

        def chunk(g, carry):
            _stage_idx(didx_all, g * CH, didx, CH)
            pltpu.sync_copy(ones_v, cntacc.at[didx], add=True)
            return carry
        lax.fori_loop(0, NCHUNK, chunk, 0)

    def rel0():
        do_rel(dst0)

    def rel1():
        do_rel(dst1)

    pl.when(c == 0)(rel0)
    pl.when(c == 1)(rel1)
    plsc.subcore_barrier()

    sl = pl.ds(s * RPS, RPS)

    @pl.when(c == 0)
    def _():
        pltpu.sync_copy(cntacc.at[sl], cnt0.at[sl])

    @pl.when(c == 1)
    def _():
        pltpu.sync_copy(cntacc.at[sl], cnt1.at[sl])


_cnt_call = pl.kernel(
    _cnt_body,
    out_type=(
        jax.ShapeDtypeStruct((NP, D), _f32),
        jax.ShapeDtypeStruct((NP, D), _f32),
    ),
    mesh=_mesh,
    scratch_types=[
        pltpu.VMEM((EPS,), _i32),
        pltpu.VMEM((CH,), _i32),
        pltpu.VMEM((CH, D), _f32),
        pltpu.VMEM_SHARED((NP, D), _f32),
    ],
)

R = 1000


def _mm_body(relu, h_ref, a0_ref, a1_ref, c0_ref, c1_ref,
             wl0_ref, wl1_ref, wr_ref, b_ref, o_ref):
    c0 = jnp.maximum(c0_ref[...], 1.0)
    c1 = jnp.maximum(c1_ref[...], 1.0)
    m0 = a0_ref[...] / c0
    m1 = a1_ref[...] / c1
    acc = jnp.dot(m0, wl0_ref[...], preferred_element_type=_f32)
    acc = acc + jnp.dot(m1, wl1_ref[...], preferred_element_type=_f32)
    acc = acc + jnp.dot(h_ref[...], wr_ref[...], preferred_element_type=_f32)
    acc = acc + b_ref[...]
    if relu:
        acc = jnp.maximum(acc, 0.0)
    o_ref[...] = acc


def _make_mm(relu):
    row_spec = pl.BlockSpec((R, D), lambda i: (i, 0))
    cnt_spec = pl.BlockSpec((R, 1), lambda i: (i, 0))
    w_spec = pl.BlockSpec((D, D), lambda i: (0, 0))
    b_spec = pl.BlockSpec((1, D), lambda i: (0, 0))
    return pl.pallas_call(
        functools.partial(_mm_body, relu),
        grid=(N // R,),
        in_specs=[row_spec, row_spec, row_spec, cnt_spec, cnt_spec,
                  w_spec, w_spec, w_spec, b_spec],
        out_specs=row_spec,
        out_shape=jax.ShapeDtypeStruct((N, D), _f32),
    )


_mm_relu = _make_mm(True)
_mm_plain = _make_mm(False)


def kernel(x, edge_index_rel0, edge_index_rel1,
           W_l_0_0, b_l_0_0, W_r_0_0, W_l_0_1, b_l_0_1, W_r_0_1,
           W_l_1_0, b_l_1_0, W_r_1_0, W_l_1_1, b_l_1_1, W_r_1_1):
    zeros = jnp.zeros((RPS, D), _f32)
    ones = jnp.ones((CH, D), _f32)

    s0, d0 = edge_index_rel0[0], edge_index_rel0[1]
    s1, d1 = edge_index_rel1[0], edge_index_rel1[1]

    cnt0, cnt1 = _cnt_call(d0, d1, zeros, ones)
    c0 = cnt0[:N, :1]
    c1 = cnt1[:N, :1]

    a0, a1 = _agg_call(x, s0, d0, s1, d1, zeros)
    h1 = _mm_relu(x, a0[:N], a1[:N], c0, c1,
                  W_l_0_0, W_l_0_1, W_r_0_0 + W_r_0_1,
                  (b_l_0_0 + b_l_0_1)[None, :])

    a0, a1 = _agg_call(h1, s0, d0, s1, d1, zeros)
    out = _mm_plain(h1, a0[:N], a1[:N], c0, c1,
                    W_l_1_0, W_l_1_1, W_r_1_0 + W_r_1_1,
                    (b_l_1_0 + b_l_1_1)[None, :])
    return out
